# baseline (speedup 1.0000x reference)
import functools

import jax
import jax.numpy as jnp
from jax import lax
from jax.experimental import pallas as pl
from jax.experimental.pallas import tpu as pltpu
from jax.experimental.pallas import tpu_sc as plsc

_PER = 100
_NCLS = 100
_BT = 128
_OUTW = 128


def _sc_row_gather(table, idx):
    n, d = table.shape
    info = plsc.get_sparse_core_info()
    nw = info.num_cores * info.num_subcores
    bpw = n // nw
    mesh = plsc.VectorSubcoreMesh(core_axis_name="c", subcore_axis_name="s")

    @functools.partial(
        pl.kernel,
        out_type=jax.ShapeDtypeStruct((n, d), table.dtype),
        mesh=mesh,
        scratch_types=[
            pltpu.VMEM((bpw,), jnp.int32),
            pltpu.VMEM((bpw, d), table.dtype),
            pltpu.SemaphoreType.DMA,
        ],
    )
    def body(table_hbm, idx_hbm, out_hbm, idx_v, rows_v, sem):
        wid = lax.axis_index("s") * info.num_cores + lax.axis_index("c")
        base = wid * bpw
        pltpu.sync_copy(idx_hbm.at[pl.ds(base, bpw)], idx_v)
        pltpu.async_copy(table_hbm.at[idx_v], rows_v, sem).wait()
        pltpu.sync_copy(rows_v, out_hbm.at[pl.ds(base, bpw)])

    return body(table, idx)


def _sc_unroute(vals, idx):
    n, d = vals.shape
    info = plsc.get_sparse_core_info()
    nw = info.num_cores * info.num_subcores
    bpw = n // nw
    mesh = plsc.VectorSubcoreMesh(core_axis_name="c", subcore_axis_name="s")

    @functools.partial(
        pl.kernel,
        out_type=jax.ShapeDtypeStruct((n, d), vals.dtype),
        mesh=mesh,
        scratch_types=[
            pltpu.VMEM((bpw,), jnp.int32),
            pltpu.VMEM((bpw, d), vals.dtype),
            pltpu.SemaphoreType.DMA,
        ],
    )
    def body(vals_hbm, idx_hbm, out_hbm, idx_v, rows_v, sem):
        wid = lax.axis_index("s") * info.num_cores + lax.axis_index("c")
        base = wid * bpw
        pltpu.sync_copy(idx_hbm.at[pl.ds(base, bpw)], idx_v)
        pltpu.sync_copy(vals_hbm.at[pl.ds(base, bpw)], rows_v)
        pltpu.async_copy(rows_v, out_hbm.at[idx_v], sem).wait()

    return body(vals, idx)


def _block_kernel(ss_ref, sc_ref, tg_ref, tgt_ref, x_ref, tw_ref, tbias_ref,
                  w_ref, bb_ref, out_ref):
    b = pl.program_id(0)
    x = x_ref[...]
    tgt = tgt_ref[...]
    cls = tgt // _PER
    within = tgt % _PER

    tl = jnp.dot(x, tw_ref[...], preferred_element_type=jnp.float32)
    tl = tl + tbias_ref[0]
    tl = tl - jnp.max(tl, axis=1, keepdims=True)
    e = jnp.exp(tl)
    sel_c = lax.broadcasted_iota(jnp.int32, e.shape, 1) == cls
    pclass = (jnp.sum(jnp.where(sel_c, e, 0.0), axis=1, keepdims=True)
              / jnp.sum(e, axis=1, keepdims=True))

    sel_w = lax.broadcasted_iota(jnp.int32, (_BT, _PER), 1) == within
    t0 = ss_ref[b]
    n = sc_ref[b]

    def chain(g):
        w = w_ref[g]
        bl = jnp.dot(x, w, preferred_element_type=jnp.float32)
        bl = bl + bb_ref[g]
        bl = bl - jnp.max(bl, axis=1, keepdims=True)
        eb = jnp.exp(bl)
        return (jnp.sum(jnp.where(sel_w, eb, 0.0), axis=1, keepdims=True)
                / jnp.sum(eb, axis=1, keepdims=True))

    def seg6(s, acc):
        gs = [tg_ref[t0 + jnp.minimum(6 * s + j, n - 1)] for j in range(6)]
        pws = [chain(g) for g in gs]
        for g, pw in zip(gs, pws):
            acc = jnp.where(cls == g, pclass * pw, acc)
        return acc

    acc = lax.fori_loop(0, (n + 5) // 6, seg6,
                        jnp.zeros((_BT, 1), jnp.float32))
    out_ref[...] = jnp.broadcast_to(acc, (_BT, _OUTW))


def _tile_metadata(scls, n_blocks):
    t_max = n_blocks + _NCLS - 1
    classes = jnp.arange(_NCLS, dtype=jnp.int32)
    starts = jnp.searchsorted(scls, classes, side='left').astype(jnp.int32)
    ends = jnp.searchsorted(scls, classes, side='right').astype(jnp.int32)
    counts = ends - starts
    block_start = starts // _BT
    block_end = jnp.where(counts > 0, (ends - 1) // _BT + 1, block_start)
    tiles_g = block_end - block_start
    tile_off = jnp.concatenate(
        [jnp.zeros((1,), jnp.int32), jnp.cumsum(tiles_g).astype(jnp.int32)])
    total = tile_off[-1]
    tids = jnp.arange(t_max, dtype=jnp.int32)
    g_of_t = jnp.clip(
        jnp.searchsorted(tile_off, tids, side='right').astype(jnp.int32) - 1,
        0, _NCLS - 1)
    b_of_t = block_start[g_of_t] + (tids - tile_off[g_of_t])
    b_of_t = jnp.clip(b_of_t, 0, n_blocks - 1)
    valid = tids < total
    tb_v = jnp.where(valid, b_of_t, n_blocks)
    blocks = jnp.arange(n_blocks, dtype=jnp.int32)
    seg_start = jnp.searchsorted(tb_v, blocks, side='left').astype(jnp.int32)
    seg_cnt = (jnp.searchsorted(tb_v, blocks, side='right').astype(jnp.int32)
               - seg_start)
    tile_group = jnp.where(valid, g_of_t, 0)
    return seg_start, seg_cnt, tile_group


def kernel(x, target, top_weights, top_bias, bottom_weights, bottom_bias):
    Bq, Tq, D = x.shape
    N = Bq * Tq
    n_blocks = N // _BT
    t_max = n_blocks + _NCLS - 1

    xb = x.reshape(N, D)
    tgt = target.reshape(N).astype(jnp.int32)
    cls = tgt // _PER

    sort_idx = jnp.argsort(cls).astype(jnp.int32)
    scls = cls[sort_idx]
    seg_start, seg_cnt, tile_group = _tile_metadata(scls, n_blocks)

    xs = _sc_row_gather(xb, sort_idx)
    tgt_s = jnp.take(tgt, sort_idx).reshape(N, 1)

    tbias = top_bias.reshape(1, 1, _NCLS)
    bb = bottom_bias.reshape(_NCLS, 1, _PER)

    grid_spec = pltpu.PrefetchScalarGridSpec(
        num_scalar_prefetch=3,
        grid=(n_blocks,),
        in_specs=[
            pl.BlockSpec((_BT, 1), lambda b, ss, sc, tg: (b, 0)),
            pl.BlockSpec((_BT, D), lambda b, ss, sc, tg: (b, 0)),
            pl.BlockSpec((D, _NCLS), lambda b, ss, sc, tg: (0, 0)),
            pl.BlockSpec((1, 1, _NCLS), lambda b, ss, sc, tg: (0, 0, 0)),
            pl.BlockSpec((_NCLS, D, _PER), lambda b, ss, sc, tg: (0, 0, 0)),
            pl.BlockSpec((_NCLS, 1, _PER), lambda b, ss, sc, tg: (0, 0, 0)),
        ],
        out_specs=pl.BlockSpec((_BT, _OUTW), lambda b, ss, sc, tg: (b, 0)),
    )
    out_s = pl.pallas_call(
        _block_kernel,
        grid_spec=grid_spec,
        out_shape=jax.ShapeDtypeStruct((N, _OUTW), jnp.float32),
        compiler_params=pltpu.CompilerParams(
            vmem_limit_bytes=100 * 1024 * 1024),
    )(seg_start, seg_cnt, tile_group, tgt_s, xs, top_weights, tbias,
      bottom_weights, bb)

    out = _sc_unroute(out_s, sort_idx)
    return out[:, :1].reshape(Bq, Tq, 1)


# speedup vs baseline: 1.0238x; 1.0238x over previous
import functools

import jax
import jax.numpy as jnp
from jax import lax
from jax.experimental import pallas as pl
from jax.experimental.pallas import tpu as pltpu
from jax.experimental.pallas import tpu_sc as plsc

_PER = 100
_NCLS = 100
_BT = 128
_OUTW = 128


def _sc_row_gather(table, idx):
    n, d = table.shape
    info = plsc.get_sparse_core_info()
    nw = info.num_cores * info.num_subcores
    bpw = n // nw
    mesh = plsc.VectorSubcoreMesh(core_axis_name="c", subcore_axis_name="s")

    @functools.partial(
        pl.kernel,
        out_type=jax.ShapeDtypeStruct((n, d), table.dtype),
        mesh=mesh,
        scratch_types=[
            pltpu.VMEM((bpw,), jnp.int32),
            pltpu.VMEM((bpw, d), table.dtype),
            pltpu.SemaphoreType.DMA,
        ],
    )
    def body(table_hbm, idx_hbm, out_hbm, idx_v, rows_v, sem):
        wid = lax.axis_index("s") * info.num_cores + lax.axis_index("c")
        base = wid * bpw
        pltpu.sync_copy(idx_hbm.at[pl.ds(base, bpw)], idx_v)
        pltpu.async_copy(table_hbm.at[idx_v], rows_v, sem).wait()
        pltpu.sync_copy(rows_v, out_hbm.at[pl.ds(base, bpw)])

    return body(table, idx)


def _sc_unroute(vals, idx):
    n, d = vals.shape
    info = plsc.get_sparse_core_info()
    nw = info.num_cores * info.num_subcores
    bpw = n // nw
    mesh = plsc.VectorSubcoreMesh(core_axis_name="c", subcore_axis_name="s")

    @functools.partial(
        pl.kernel,
        out_type=jax.ShapeDtypeStruct((n, d), vals.dtype),
        mesh=mesh,
        scratch_types=[
            pltpu.VMEM((bpw,), jnp.int32),
            pltpu.VMEM((bpw, d), vals.dtype),
            pltpu.SemaphoreType.DMA,
        ],
    )
    def body(vals_hbm, idx_hbm, out_hbm, idx_v, rows_v, sem):
        wid = lax.axis_index("s") * info.num_cores + lax.axis_index("c")
        base = wid * bpw
        pltpu.sync_copy(idx_hbm.at[pl.ds(base, bpw)], idx_v)
        pltpu.sync_copy(vals_hbm.at[pl.ds(base, bpw)], rows_v)
        pltpu.async_copy(rows_v, out_hbm.at[idx_v], sem).wait()

    return body(vals, idx)


def _block_kernel(ss_ref, sc_ref, tg_ref, tgt_ref, x_ref, tw_ref, tbias_ref,
                  w_ref, bb_ref, out_ref):
    b = pl.program_id(0)
    x = x_ref[...]
    tgt = tgt_ref[...]
    cls = tgt // _PER
    within = tgt % _PER

    tl = jnp.dot(x, tw_ref[...], preferred_element_type=jnp.float32)
    tl = tl + tbias_ref[0]
    tl = tl - jnp.max(tl, axis=1, keepdims=True)
    e = jnp.exp(tl)
    sel_c = lax.broadcasted_iota(jnp.int32, e.shape, 1) == cls
    pclass = (jnp.sum(jnp.where(sel_c, e, 0.0), axis=1, keepdims=True)
              / jnp.sum(e, axis=1, keepdims=True))

    sel_w = lax.broadcasted_iota(jnp.int32, (_BT, _PER), 1) == within
    t0 = ss_ref[b]
    n = sc_ref[b]

    def chain(g):
        w = w_ref[g]
        bl = jnp.dot(x, w, preferred_element_type=jnp.float32)
        bl = bl + bb_ref[g]
        bl = bl - jnp.max(bl, axis=1, keepdims=True)
        eb = jnp.exp(bl)
        return (jnp.sum(jnp.where(sel_w, eb, 0.0), axis=1, keepdims=True)
                / jnp.sum(eb, axis=1, keepdims=True))

    def seg4(s, acc):
        g1 = tg_ref[t0 + 4 * s]
        g2 = tg_ref[t0 + jnp.minimum(4 * s + 1, n - 1)]
        g3 = tg_ref[t0 + jnp.minimum(4 * s + 2, n - 1)]
        g4 = tg_ref[t0 + jnp.minimum(4 * s + 3, n - 1)]
        pw1 = chain(g1)
        pw2 = chain(g2)
        pw3 = chain(g3)
        pw4 = chain(g4)
        acc = jnp.where(cls == g1, pclass * pw1, acc)
        acc = jnp.where(cls == g2, pclass * pw2, acc)
        acc = jnp.where(cls == g3, pclass * pw3, acc)
        return jnp.where(cls == g4, pclass * pw4, acc)

    acc = lax.fori_loop(0, (n + 3) // 4, seg4,
                        jnp.zeros((_BT, 1), jnp.float32))
    out_ref[...] = jnp.broadcast_to(acc, (_BT, _OUTW))


def _tile_metadata(scls, n_blocks):
    t_max = n_blocks + _NCLS - 1
    classes = jnp.arange(_NCLS, dtype=jnp.int32)
    starts = jnp.searchsorted(scls, classes, side='left').astype(jnp.int32)
    ends = jnp.searchsorted(scls, classes, side='right').astype(jnp.int32)
    counts = ends - starts
    block_start = starts // _BT
    block_end = jnp.where(counts > 0, (ends - 1) // _BT + 1, block_start)
    tiles_g = block_end - block_start
    tile_off = jnp.concatenate(
        [jnp.zeros((1,), jnp.int32), jnp.cumsum(tiles_g).astype(jnp.int32)])
    total = tile_off[-1]
    tids = jnp.arange(t_max, dtype=jnp.int32)
    g_of_t = jnp.clip(
        jnp.searchsorted(tile_off, tids, side='right').astype(jnp.int32) - 1,
        0, _NCLS - 1)
    b_of_t = block_start[g_of_t] + (tids - tile_off[g_of_t])
    b_of_t = jnp.clip(b_of_t, 0, n_blocks - 1)
    valid = tids < total
    tb_v = jnp.where(valid, b_of_t, n_blocks)
    blocks = jnp.arange(n_blocks, dtype=jnp.int32)
    seg_start = jnp.searchsorted(tb_v, blocks, side='left').astype(jnp.int32)
    seg_cnt = (jnp.searchsorted(tb_v, blocks, side='right').astype(jnp.int32)
               - seg_start)
    tile_group = jnp.where(valid, g_of_t, 0)
    return seg_start, seg_cnt, tile_group


def kernel(x, target, top_weights, top_bias, bottom_weights, bottom_bias):
    Bq, Tq, D = x.shape
    N = Bq * Tq
    n_blocks = N // _BT
    t_max = n_blocks + _NCLS - 1

    xb = x.reshape(N, D)
    tgt = target.reshape(N).astype(jnp.int32)
    cls = tgt // _PER

    sort_idx = jnp.argsort(cls).astype(jnp.int32)
    scls = cls[sort_idx]
    seg_start, seg_cnt, tile_group = _tile_metadata(scls, n_blocks)

    xs = _sc_row_gather(xb, sort_idx)
    tgt_s = jnp.take(tgt, sort_idx).reshape(N, 1)

    tbias = top_bias.reshape(1, 1, _NCLS)
    bb = bottom_bias.reshape(_NCLS, 1, _PER)

    grid_spec = pltpu.PrefetchScalarGridSpec(
        num_scalar_prefetch=3,
        grid=(n_blocks,),
        in_specs=[
            pl.BlockSpec((_BT, 1), lambda b, ss, sc, tg: (b, 0)),
            pl.BlockSpec((_BT, D), lambda b, ss, sc, tg: (b, 0)),
            pl.BlockSpec((D, _NCLS), lambda b, ss, sc, tg: (0, 0)),
            pl.BlockSpec((1, 1, _NCLS), lambda b, ss, sc, tg: (0, 0, 0)),
            pl.BlockSpec((_NCLS, D, _PER), lambda b, ss, sc, tg: (0, 0, 0)),
            pl.BlockSpec((_NCLS, 1, _PER), lambda b, ss, sc, tg: (0, 0, 0)),
        ],
        out_specs=pl.BlockSpec((_BT, _OUTW), lambda b, ss, sc, tg: (b, 0)),
    )
    out_s = pl.pallas_call(
        _block_kernel,
        grid_spec=grid_spec,
        out_shape=jax.ShapeDtypeStruct((N, _OUTW), jnp.float32),
        compiler_params=pltpu.CompilerParams(
            vmem_limit_bytes=100 * 1024 * 1024),
    )(seg_start, seg_cnt, tile_group, tgt_s, xs, top_weights, tbias,
      bottom_weights, bb)

    out = _sc_unroute(out_s, sort_idx)
    return out[:, :1].reshape(Bq, Tq, 1)
